# Initial kernel scaffold; baseline (speedup 1.0000x reference)
import jax
import jax.numpy as jnp
from jax.experimental import pallas as pl


def kernel(cls_score_0, bbox_pred_0, cls_score_1, bbox_pred_1, cls_score_2, bbox_pred_2, cls_score_3, bbox_pred_3, cls_score_4, bbox_pred_4, cls_score_5, bbox_pred_5):
    raise NotImplementedError("write your pallas kernel here")


# speedup vs baseline: 47.2918x; 47.2918x over previous
import numpy as np
import jax
import jax.numpy as jnp
from jax.experimental import pallas as pl
from jax.experimental.pallas import tpu as pltpu

_STRIDES = [16, 32, 64, 107, 160, 320]
_MIN_SIZES = [48, 100, 150, 202, 253, 304]
_MAX_SIZES = [100, 150, 202, 253, 304, 320]
_FEAT_SIZES = [(20, 20), (10, 10), (5, 5), (3, 3), (2, 2), (1, 1)]
_NC = 80
_SCORE_THR = 0.02
_NMS_PRE = 1000
_IOU_THR = 0.45
_MAX_OUT = 200
_MAX_RATIO = float(abs(np.log(0.016)))

_NANCH = [fh * fw * 6 for fh, fw in _FEAT_SIZES]
_OFFS = np.concatenate([[0], np.cumsum(_NANCH)])
_ATOT = int(sum(_NANCH))
_APAD = 3328
_KS = [min(_NMS_PRE, n * _NC) for n in _NANCH]
_M = int(sum(_KS))
_ROWS, _LANES = 8, 704
_MP = _ROWS * _LANES


def _priors_np():
    ratios = np.array([1.0, 0.5, 2.0, 1.0 / 3.0, 3.0])
    outs = []
    for k in range(6):
        base = float(_MIN_SIZES[k])
        scales = np.array([1.0, np.sqrt(_MAX_SIZES[k] / _MIN_SIZES[k])])
        cx = cy = _STRIDES[k] / 2.0
        h_r = np.sqrt(ratios)
        w_r = 1.0 / h_r
        ws = (base * scales[:, None] * w_r[None, :]).reshape(-1)
        hs = (base * scales[:, None] * h_r[None, :]).reshape(-1)
        ba = np.stack([cx - 0.5 * ws, cy - 0.5 * hs, cx + 0.5 * ws, cy + 0.5 * hs], -1)
        ba = ba[[0, 5, 1, 2, 3, 4]].astype(np.float32)
        fh, fw = _FEAT_SIZES[k]
        s = float(_STRIDES[k])
        xx = np.tile(np.arange(fw, dtype=np.float32) * s, fh)
        yy = np.repeat(np.arange(fh, dtype=np.float32) * s, fw)
        shifts = np.stack([xx, yy, xx, yy], axis=-1)
        outs.append((ba[None, :, :] + shifts[:, None, :]).reshape(-1, 4))
    return np.concatenate(outs, axis=0).astype(np.float32)


_PRIORS = _priors_np()


def _score_decode_kernel(lg_ref, dx_ref, dy_ref, dw_ref, dh_ref,
                         px1_ref, py1_ref, px2_ref, py2_ref,
                         ms_ref, bx1_ref, by1_ref, bx2_ref, by2_ref):
    x = lg_ref[...]
    lane = jax.lax.broadcasted_iota(jnp.int32, x.shape, 1)
    neg = jnp.float32(-jnp.inf)
    m = jnp.max(jnp.where(lane < 81, x, neg), axis=-1, keepdims=True)
    e = jnp.where(lane < 81, jnp.exp(x - m), 0.0)
    s = e / jnp.sum(e, axis=-1, keepdims=True)
    ms_ref[...] = jnp.where((lane < _NC) & (s > _SCORE_THR), s, neg)

    dx = dx_ref[...] * 0.1
    dy = dy_ref[...] * 0.1
    dw = jnp.clip(dw_ref[...] * 0.2, -_MAX_RATIO, _MAX_RATIO)
    dh = jnp.clip(dh_ref[...] * 0.2, -_MAX_RATIO, _MAX_RATIO)
    px1, py1, px2, py2 = px1_ref[...], py1_ref[...], px2_ref[...], py2_ref[...]
    pw = px2 - px1
    ph = py2 - py1
    cx = (px1 + px2) * 0.5
    cy = (py1 + py2) * 0.5
    gx = cx + pw * dx
    gy = cy + ph * dy
    gw = pw * jnp.exp(dw)
    gh = ph * jnp.exp(dh)
    bx1_ref[...] = jnp.clip(gx - gw * 0.5, 0.0, 320.0)
    by1_ref[...] = jnp.clip(gy - gh * 0.5, 0.0, 320.0)
    bx2_ref[...] = jnp.clip(gx + gw * 0.5, 0.0, 320.0)
    by2_ref[...] = jnp.clip(gy + gh * 0.5, 0.0, 320.0)


def _nms_kernel(bx1_ref, by1_ref, bx2_ref, by2_ref, lab_ref, sc_ref, out_ref,
                x1s, y1s, x2s, y2s, area_s, supp_s):
    X1 = bx1_ref[0]
    Y1 = by1_ref[0]
    X2 = bx2_ref[0]
    Y2 = by2_ref[0]
    LF = lab_ref[0]
    SC = sc_ref[0]
    neg = jnp.float32(-jnp.inf)
    V = SC > _SCORE_THR
    raw = jnp.maximum(
        jnp.max(jnp.where(V, X1, neg)),
        jnp.maximum(jnp.max(jnp.where(V, Y1, neg)),
                    jnp.maximum(jnp.max(jnp.where(V, X2, neg)),
                                jnp.max(jnp.where(V, Y2, neg)))))
    any_v = jnp.any(V)
    offv = jnp.where(any_v, raw, jnp.float32(0.0)) + 1.0

    x1s[...] = X1 + LF * offv
    y1s[...] = Y1 + LF * offv
    x2s[...] = X2 + LF * offv
    y2s[...] = Y2 + LF * offv
    area_s[...] = (X2 - X1) * (Y2 - Y1)
    supp_s[...] = jnp.zeros((_ROWS, _LANES), jnp.float32)

    sub = jax.lax.broadcasted_iota(jnp.int32, (_ROWS, _LANES), 0)
    lane = jax.lax.broadcasted_iota(jnp.int32, (_ROWS, _LANES), 1)
    fidx = sub * _LANES + lane

    l2 = jax.lax.broadcasted_iota(jnp.int32, (208, 128), 1)
    s2 = jax.lax.broadcasted_iota(jnp.int32, (208, 128), 0)
    out_ref[0] = jnp.where(l2 == 5, jnp.float32(-1.0), jnp.float32(0.0))

    zero = jnp.float32(0.0)

    def cond(st):
        i, nk = st
        return (i < _M) & (nk < _MAX_OUT)

    def body(st):
        i, nk = st
        msk = fidx == i
        sci = jnp.sum(jnp.where(msk, SC, zero))
        vi = sci > _SCORE_THR
        si = jnp.sum(jnp.where(msk, supp_s[...], zero))
        keep = vi & (si == 0.0)

        ox1 = jnp.sum(jnp.where(msk, X1, zero))
        oy1 = jnp.sum(jnp.where(msk, Y1, zero))
        ox2 = jnp.sum(jnp.where(msk, X2, zero))
        oy2 = jnp.sum(jnp.where(msk, Y2, zero))
        lbi = jnp.sum(jnp.where(msk, LF, zero))
        off = lbi * offv
        px1 = ox1 + off
        py1 = oy1 + off
        px2 = ox2 + off
        py2 = oy2 + off

        xx1 = jnp.maximum(px1, x1s[...])
        yy1 = jnp.maximum(py1, y1s[...])
        xx2 = jnp.minimum(px2, x2s[...])
        yy2 = jnp.minimum(py2, y2s[...])
        inter = jnp.maximum(0.0, xx2 - xx1) * jnp.maximum(0.0, yy2 - yy1)
        ai = (px2 - px1) * (py2 - py1)
        iou = inter / (ai + area_s[...] - inter + 1e-12)
        supp_s[...] = jnp.where(keep & (iou > _IOU_THR) & (fidx > i),
                                jnp.float32(1.0), supp_s[...])

        @pl.when(keep)
        def _():
            row = jnp.where(l2 == 0, ox1,
                  jnp.where(l2 == 1, oy1,
                  jnp.where(l2 == 2, ox2,
                  jnp.where(l2 == 3, oy2,
                  jnp.where(l2 == 4, sci,
                  jnp.where(l2 == 5, lbi, zero))))))
            out_ref[0] = jnp.where(s2 == nk, row, out_ref[0])

        i_next = jnp.where(vi, i + 1, _M)
        nk_next = nk + jnp.where(keep, 1, 0)
        return (i_next, nk_next)

    jax.lax.while_loop(cond, body, (jnp.int32(0), jnp.int32(0)))


def kernel(cls_score_0, bbox_pred_0, cls_score_1, bbox_pred_1,
           cls_score_2, bbox_pred_2, cls_score_3, bbox_pred_3,
           cls_score_4, bbox_pred_4, cls_score_5, bbox_pred_5):
    cls = [cls_score_0, cls_score_1, cls_score_2, cls_score_3, cls_score_4, cls_score_5]
    bps = [bbox_pred_0, bbox_pred_1, bbox_pred_2, bbox_pred_3, bbox_pred_4, bbox_pred_5]
    B = cls[0].shape[0]

    logits = jnp.concatenate(
        [jnp.transpose(c, (0, 2, 3, 1)).reshape(B, -1, _NC + 1) for c in cls], axis=1)
    logits = jnp.pad(logits, ((0, 0), (0, 0), (0, 128 - (_NC + 1)))).reshape(B * _ATOT, 128)

    deltas = jnp.concatenate(
        [jnp.transpose(p, (0, 2, 3, 1)).reshape(B, -1, 4) for p in bps], axis=1)
    deltas = jnp.pad(deltas, ((0, 0), (0, _APAD - _ATOT), (0, 0)))
    dcomp = [deltas[:, :, j].reshape(B * (_APAD // 128), 128) for j in range(4)]

    prn = np.pad(_PRIORS, ((0, _APAD - _ATOT), (0, 0)))
    pcomp = [jnp.asarray(np.broadcast_to(prn[None, :, j], (B, _APAD))
                         .reshape(B * (_APAD // 128), 128)) for j in range(4)]

    f32 = jnp.float32
    ms, bx1a, by1a, bx2a, by2a = pl.pallas_call(
        _score_decode_kernel,
        out_shape=[
            jax.ShapeDtypeStruct((B * _ATOT, 128), f32),
            jax.ShapeDtypeStruct((B * (_APAD // 128), 128), f32),
            jax.ShapeDtypeStruct((B * (_APAD // 128), 128), f32),
            jax.ShapeDtypeStruct((B * (_APAD // 128), 128), f32),
            jax.ShapeDtypeStruct((B * (_APAD // 128), 128), f32),
        ],
    )(logits, *dcomp, *pcomp)

    ms = ms.reshape(B, _ATOT, 128)[:, :, :_NC]
    boxc = [a.reshape(B, _APAD) for a in (bx1a, by1a, bx2a, by2a)]

    vals_l, labs_l = [], []
    sel = [[], [], [], []]
    for lvl in range(6):
        o, n, k = int(_OFFS[lvl]), _NANCH[lvl], _KS[lvl]
        flat = ms[:, o:o + n, :].reshape(B, n * _NC)
        v, idx = jax.lax.top_k(flat, k)
        a = o + idx // _NC
        vals_l.append(v)
        labs_l.append(idx % _NC)
        for j in range(4):
            sel[j].append(jnp.take_along_axis(boxc[j], a, axis=1))

    scores = jnp.concatenate(vals_l, axis=1)
    labels = jnp.concatenate(labs_l, axis=1)
    comps = [jnp.concatenate(s, axis=1) for s in sel]

    gorder = jnp.argsort(-scores, axis=1)
    scores = jnp.take_along_axis(scores, gorder, axis=1)
    labels = jnp.take_along_axis(labels, gorder, axis=1)
    comps = [jnp.take_along_axis(cpt, gorder, axis=1) for cpt in comps]

    pad = _MP - _M
    scores = jnp.pad(scores, ((0, 0), (0, pad)), constant_values=-jnp.inf)
    labf = jnp.pad(labels.astype(f32), ((0, 0), (0, pad)))
    comps = [jnp.pad(cpt, ((0, 0), (0, pad))) for cpt in comps]

    shp = (B, _ROWS, _LANES)
    vecs = [a.reshape(shp) for a in comps + [labf, scores]]

    out = pl.pallas_call(
        _nms_kernel,
        grid=(B,),
        in_specs=[pl.BlockSpec((1, _ROWS, _LANES), lambda b: (b, 0, 0))] * 6,
        out_specs=pl.BlockSpec((1, 208, 128), lambda b: (b, 0, 0)),
        out_shape=jax.ShapeDtypeStruct((B, 208, 128), f32),
        scratch_shapes=[pltpu.VMEM((_ROWS, _LANES), f32)] * 6,
    )(*vecs)

    dets = out[:, :_MAX_OUT, :5]
    labs = out[:, :_MAX_OUT, 5].astype(jnp.int32)
    return dets, labs


# speedup vs baseline: 51.3511x; 1.0858x over previous
import numpy as np
import jax
import jax.numpy as jnp
from jax.experimental import pallas as pl
from jax.experimental.pallas import tpu as pltpu

_STRIDES = [16, 32, 64, 107, 160, 320]
_MIN_SIZES = [48, 100, 150, 202, 253, 304]
_MAX_SIZES = [100, 150, 202, 253, 304, 320]
_FEAT_SIZES = [(20, 20), (10, 10), (5, 5), (3, 3), (2, 2), (1, 1)]
_NC = 80
_SCORE_THR = 0.02
_NMS_PRE = 1000
_IOU_THR = 0.45
_MAX_OUT = 200
_MAX_RATIO = float(abs(np.log(0.016)))

_NANCH = [fh * fw * 6 for fh, fw in _FEAT_SIZES]
_OFFS = np.concatenate([[0], np.cumsum(_NANCH)])
_ATOT = int(sum(_NANCH))
_APAD = 3328
_KS = [min(_NMS_PRE, n * _NC) for n in _NANCH]
_M = int(sum(_KS))
_ROWS, _LANES = 8, 704
_MP = _ROWS * _LANES


def _priors_np():
    ratios = np.array([1.0, 0.5, 2.0, 1.0 / 3.0, 3.0])
    outs = []
    for k in range(6):
        base = float(_MIN_SIZES[k])
        scales = np.array([1.0, np.sqrt(_MAX_SIZES[k] / _MIN_SIZES[k])])
        cx = cy = _STRIDES[k] / 2.0
        h_r = np.sqrt(ratios)
        w_r = 1.0 / h_r
        ws = (base * scales[:, None] * w_r[None, :]).reshape(-1)
        hs = (base * scales[:, None] * h_r[None, :]).reshape(-1)
        ba = np.stack([cx - 0.5 * ws, cy - 0.5 * hs, cx + 0.5 * ws, cy + 0.5 * hs], -1)
        ba = ba[[0, 5, 1, 2, 3, 4]].astype(np.float32)
        fh, fw = _FEAT_SIZES[k]
        s = float(_STRIDES[k])
        xx = np.tile(np.arange(fw, dtype=np.float32) * s, fh)
        yy = np.repeat(np.arange(fh, dtype=np.float32) * s, fw)
        shifts = np.stack([xx, yy, xx, yy], axis=-1)
        outs.append((ba[None, :, :] + shifts[:, None, :]).reshape(-1, 4))
    return np.concatenate(outs, axis=0).astype(np.float32)


_PRIORS = _priors_np()


def _score_decode_kernel(lg_ref, dx_ref, dy_ref, dw_ref, dh_ref,
                         px1_ref, py1_ref, px2_ref, py2_ref,
                         ms_ref, bx1_ref, by1_ref, bx2_ref, by2_ref):
    x = lg_ref[...]
    lane = jax.lax.broadcasted_iota(jnp.int32, x.shape, 1)
    neg = jnp.float32(-jnp.inf)
    m = jnp.max(jnp.where(lane < 81, x, neg), axis=-1, keepdims=True)
    e = jnp.where(lane < 81, jnp.exp(x - m), 0.0)
    s = e / jnp.sum(e, axis=-1, keepdims=True)
    ms_ref[...] = jnp.where((lane < _NC) & (s > _SCORE_THR), s, neg)

    dx = dx_ref[...] * 0.1
    dy = dy_ref[...] * 0.1
    dw = jnp.clip(dw_ref[...] * 0.2, -_MAX_RATIO, _MAX_RATIO)
    dh = jnp.clip(dh_ref[...] * 0.2, -_MAX_RATIO, _MAX_RATIO)
    px1, py1, px2, py2 = px1_ref[...], py1_ref[...], px2_ref[...], py2_ref[...]
    pw = px2 - px1
    ph = py2 - py1
    cx = (px1 + px2) * 0.5
    cy = (py1 + py2) * 0.5
    gx = cx + pw * dx
    gy = cy + ph * dy
    gw = pw * jnp.exp(dw)
    gh = ph * jnp.exp(dh)
    bx1_ref[...] = jnp.clip(gx - gw * 0.5, 0.0, 320.0)
    by1_ref[...] = jnp.clip(gy - gh * 0.5, 0.0, 320.0)
    bx2_ref[...] = jnp.clip(gx + gw * 0.5, 0.0, 320.0)
    by2_ref[...] = jnp.clip(gy + gh * 0.5, 0.0, 320.0)


def _nms_kernel(bx1_ref, by1_ref, bx2_ref, by2_ref, lab_ref, sc_ref, out_ref,
                x1s, y1s, x2s, y2s, area_s, supp_s):
    X1 = bx1_ref[0]
    Y1 = by1_ref[0]
    X2 = bx2_ref[0]
    Y2 = by2_ref[0]
    LF = lab_ref[0]
    SC = sc_ref[0]
    neg = jnp.float32(-jnp.inf)
    V = SC > _SCORE_THR
    raw = jnp.maximum(
        jnp.max(jnp.where(V, X1, neg)),
        jnp.maximum(jnp.max(jnp.where(V, Y1, neg)),
                    jnp.maximum(jnp.max(jnp.where(V, X2, neg)),
                                jnp.max(jnp.where(V, Y2, neg)))))
    any_v = jnp.any(V)
    offv = jnp.where(any_v, raw, jnp.float32(0.0)) + 1.0

    x1s[...] = X1 + LF * offv
    y1s[...] = Y1 + LF * offv
    x2s[...] = X2 + LF * offv
    y2s[...] = Y2 + LF * offv
    area_s[...] = (X2 - X1) * (Y2 - Y1)
    supp_s[...] = jnp.zeros((_ROWS, _LANES), jnp.float32)

    sub = jax.lax.broadcasted_iota(jnp.int32, (_ROWS, _LANES), 0)
    lane = jax.lax.broadcasted_iota(jnp.int32, (_ROWS, _LANES), 1)
    fidx = sub * _LANES + lane

    l2 = jax.lax.broadcasted_iota(jnp.int32, (208, 128), 1)
    s2 = jax.lax.broadcasted_iota(jnp.int32, (208, 128), 0)
    out_ref[0] = jnp.where(l2 == 5, jnp.float32(-1.0), jnp.float32(0.0))

    zero = jnp.float32(0.0)

    def cond(st):
        i, nk = st
        return (i < _M) & (nk < _MAX_OUT)

    def body(st):
        i, nk = st
        msk = fidx == i
        sci = jnp.sum(jnp.where(msk, SC, zero))
        vi = sci > _SCORE_THR
        si = jnp.sum(jnp.where(msk, supp_s[...], zero))
        keep = vi & (si == 0.0)

        ox1 = jnp.sum(jnp.where(msk, X1, zero))
        oy1 = jnp.sum(jnp.where(msk, Y1, zero))
        ox2 = jnp.sum(jnp.where(msk, X2, zero))
        oy2 = jnp.sum(jnp.where(msk, Y2, zero))
        lbi = jnp.sum(jnp.where(msk, LF, zero))
        off = lbi * offv
        px1 = ox1 + off
        py1 = oy1 + off
        px2 = ox2 + off
        py2 = oy2 + off

        xx1 = jnp.maximum(px1, x1s[...])
        yy1 = jnp.maximum(py1, y1s[...])
        xx2 = jnp.minimum(px2, x2s[...])
        yy2 = jnp.minimum(py2, y2s[...])
        inter = jnp.maximum(0.0, xx2 - xx1) * jnp.maximum(0.0, yy2 - yy1)
        ai = (px2 - px1) * (py2 - py1)
        iou = inter / (ai + area_s[...] - inter + 1e-12)
        supp_s[...] = jnp.where(keep & (iou > _IOU_THR) & (fidx > i),
                                jnp.float32(1.0), supp_s[...])

        @pl.when(keep)
        def _():
            row = jnp.where(l2 == 0, ox1,
                  jnp.where(l2 == 1, oy1,
                  jnp.where(l2 == 2, ox2,
                  jnp.where(l2 == 3, oy2,
                  jnp.where(l2 == 4, sci,
                  jnp.where(l2 == 5, lbi, zero))))))
            out_ref[0] = jnp.where(s2 == nk, row, out_ref[0])

        i_next = jnp.where(vi, i + 1, _M)
        nk_next = nk + jnp.where(keep, 1, 0)
        return (i_next, nk_next)

    jax.lax.while_loop(cond, body, (jnp.int32(0), jnp.int32(0)))


def kernel(cls_score_0, bbox_pred_0, cls_score_1, bbox_pred_1,
           cls_score_2, bbox_pred_2, cls_score_3, bbox_pred_3,
           cls_score_4, bbox_pred_4, cls_score_5, bbox_pred_5):
    cls = [cls_score_0, cls_score_1, cls_score_2, cls_score_3, cls_score_4, cls_score_5]
    bps = [bbox_pred_0, bbox_pred_1, bbox_pred_2, bbox_pred_3, bbox_pred_4, bbox_pred_5]
    B = cls[0].shape[0]

    logits = jnp.concatenate(
        [jnp.transpose(c, (0, 2, 3, 1)).reshape(B, -1, _NC + 1) for c in cls], axis=1)
    logits = jnp.pad(logits, ((0, 0), (0, 0), (0, 128 - (_NC + 1)))).reshape(B * _ATOT, 128)

    deltas = jnp.concatenate(
        [jnp.transpose(p, (0, 2, 3, 1)).reshape(B, -1, 4) for p in bps], axis=1)
    deltas = jnp.pad(deltas, ((0, 0), (0, _APAD - _ATOT), (0, 0)))
    dcomp = [deltas[:, :, j].reshape(B * (_APAD // 128), 128) for j in range(4)]

    prn = np.pad(_PRIORS, ((0, _APAD - _ATOT), (0, 0)))
    pcomp = [jnp.asarray(np.broadcast_to(prn[None, :, j], (B, _APAD))
                         .reshape(B * (_APAD // 128), 128)) for j in range(4)]

    f32 = jnp.float32
    ms, bx1a, by1a, bx2a, by2a = pl.pallas_call(
        _score_decode_kernel,
        out_shape=[
            jax.ShapeDtypeStruct((B * _ATOT, 128), f32),
            jax.ShapeDtypeStruct((B * (_APAD // 128), 128), f32),
            jax.ShapeDtypeStruct((B * (_APAD // 128), 128), f32),
            jax.ShapeDtypeStruct((B * (_APAD // 128), 128), f32),
            jax.ShapeDtypeStruct((B * (_APAD // 128), 128), f32),
        ],
    )(logits, *dcomp, *pcomp)

    ms = ms.reshape(B, _ATOT, 128)[:, :, :_NC]
    boxes_all = jnp.stack(
        [a.reshape(B, _APAD) for a in (bx1a, by1a, bx2a, by2a)], axis=-1)

    vals_l, pidx_l = [], []
    for lvl in range(6):
        o, n, k = int(_OFFS[lvl]), _NANCH[lvl], _KS[lvl]
        flat = ms[:, o:o + n, :].reshape(B, n * _NC)
        if k == n * _NC:
            v = flat
            idx = jnp.broadcast_to(jnp.arange(k, dtype=jnp.int32), (B, k))
        else:
            v, idx = jax.lax.top_k(flat, k)
        vals_l.append(v)
        pidx_l.append(_NC * o + idx)

    scores = jnp.concatenate(vals_l, axis=1)
    pidx = jnp.concatenate(pidx_l, axis=1)

    gorder = jnp.argsort(-scores, axis=1)
    scores = jnp.take_along_axis(scores, gorder, axis=1)
    pidx = jnp.take_along_axis(pidx, gorder, axis=1)
    labels = pidx % _NC
    agl = pidx // _NC
    sboxes = jnp.take_along_axis(boxes_all, agl[:, :, None], axis=1)

    pad = _MP - _M
    scores = jnp.pad(scores, ((0, 0), (0, pad)), constant_values=-jnp.inf)
    labf = jnp.pad(labels.astype(f32), ((0, 0), (0, pad)))
    sboxes = jnp.pad(sboxes, ((0, 0), (0, pad), (0, 0)))

    shp = (B, _ROWS, _LANES)
    vecs = [sboxes[:, :, j].reshape(shp) for j in range(4)]
    vecs += [labf.reshape(shp), scores.reshape(shp)]

    out = pl.pallas_call(
        _nms_kernel,
        grid=(B,),
        in_specs=[pl.BlockSpec((1, _ROWS, _LANES), lambda b: (b, 0, 0))] * 6,
        out_specs=pl.BlockSpec((1, 208, 128), lambda b: (b, 0, 0)),
        out_shape=jax.ShapeDtypeStruct((B, 208, 128), f32),
        scratch_shapes=[pltpu.VMEM((_ROWS, _LANES), f32)] * 6,
    )(*vecs)

    dets = out[:, :_MAX_OUT, :5]
    labs = out[:, :_MAX_OUT, 5].astype(jnp.int32)
    return dets, labs
